# baseline (speedup 1.0000x reference)
import functools

import jax
import jax.numpy as jnp
from jax import lax
from jax.experimental import pallas as pl
from jax.experimental.pallas import tpu as pltpu
from jax.experimental.pallas import tpu_sc as plsc

_MF_DIM = 32
_HIDDEN = 64


def _make_sc_gather(batch, dim):
    info = plsc.get_sparse_core_info()
    nc, ns = info.num_cores, info.num_subcores
    nw = nc * ns
    assert batch % (8 * nw) == 0
    b_per_w = batch // nw
    mesh = plsc.VectorSubcoreMesh(core_axis_name="c", subcore_axis_name="s")

    @functools.partial(
        pl.kernel,
        mesh=mesh,
        compiler_params=pltpu.CompilerParams(use_tc_tiling_on_sc=False),
        out_type=(
            jax.ShapeDtypeStruct((batch, dim), jnp.float32),
            jax.ShapeDtypeStruct((batch, dim), jnp.float32),
        ),
        scratch_types=[
            pltpu.VMEM((b_per_w,), jnp.int32),
            pltpu.VMEM((b_per_w,), jnp.int32),
            pltpu.VMEM((b_per_w, dim), jnp.float32),
            pltpu.VMEM((b_per_w, dim), jnp.float32),
            pltpu.SemaphoreType.DMA,
            pltpu.SemaphoreType.DMA,
        ],
    )
    def gather_kernel(user_hbm, item_hbm, utab_hbm, itab_hbm, uout_hbm,
                      iout_hbm, uidx_v, iidx_v, urows_v, irows_v, usem, isem):
        wid = lax.axis_index("s") * nc + lax.axis_index("c")
        base = wid * b_per_w
        pltpu.sync_copy(user_hbm.at[pl.ds(base, b_per_w)], uidx_v)
        pltpu.sync_copy(item_hbm.at[pl.ds(base, b_per_w)], iidx_v)
        ucp = pltpu.async_copy(utab_hbm.at[uidx_v], urows_v, usem)
        icp = pltpu.async_copy(itab_hbm.at[iidx_v], irows_v, isem)
        ucp.wait()
        icp.wait()
        pltpu.sync_copy(urows_v, uout_hbm.at[pl.ds(base, b_per_w)])
        pltpu.sync_copy(irows_v, iout_hbm.at[pl.ds(base, b_per_w)])

    return gather_kernel


def _tc_body(u_ref, i_ref, w_ref, b_ref, o_ref):
    s = u_ref[...] + i_ref[...]
    o_ref[...] = jnp.tanh(
        jnp.dot(s, w_ref[...], preferred_element_type=jnp.float32)
        + b_ref[...]
    )


def kernel(user, item, user_table, item_table, W, b):
    batch = user.shape[0]
    gather = _make_sc_gather(batch, _MF_DIM)
    user_embed, item_embed = gather(user, item, user_table, item_table)

    blk = 2048
    hidden = pl.pallas_call(
        _tc_body,
        grid=(batch // blk,),
        in_specs=[
            pl.BlockSpec((blk, _MF_DIM), lambda i: (i, 0)),
            pl.BlockSpec((blk, _MF_DIM), lambda i: (i, 0)),
            pl.BlockSpec((_MF_DIM, _HIDDEN), lambda i: (0, 0)),
            pl.BlockSpec((1, _HIDDEN), lambda i: (0, 0)),
        ],
        out_specs=pl.BlockSpec((blk, _HIDDEN), lambda i: (i, 0)),
        out_shape=jax.ShapeDtypeStruct((batch, _HIDDEN), jnp.float32),
    )(user_embed, item_embed, W.T, b.reshape(1, _HIDDEN))

    return hidden.reshape(1, batch, _HIDDEN), user_embed, item_embed
